# baseline (speedup 1.0000x reference)
import jax
import jax.numpy as jnp
from jax import lax
from jax.experimental import pallas as pl
from jax.experimental.pallas import tpu as pltpu
from jax.experimental.pallas import tpu_sc as plsc

_STRIDE = 2
_B, _H, _W, _C = 8, 112, 112, 96
_N = _H * _W * _C
_M = _N * _STRIDE * _STRIDE
_NC, _NS, _L = 2, 16, 16
_NQ = 4
_CH = _M // _NQ
_TRASH = 4096
_P = _N // _NS
_NSTEP = 7
_K = _P // _NSTEP
_STRIPE = _CH // _NS
_NZ = 7
_ZB = _STRIPE // _NZ
_BPC = _B // _NC


def _unpool_body(x_hbm, m_hbm, out_hbm, mask_v, vals_v, idx_v, zero_v, acc_sh):
    cid = lax.axis_index("c")
    sid = lax.axis_index("s")

    z16 = jnp.zeros((_L,), jnp.float32)

    @pl.loop(0, _ZB // _L)
    def _(i):
        zero_v[pl.ds(i * _L, _L)] = z16

    ch_vec = jnp.full((_L,), _CH, jnp.uint32)
    tmask_vec = jnp.full((_L,), _TRASH - 1, jnp.uint32)

    @pl.loop(0, _BPC)
    def _(b_loc):
        b = cid * _BPC + b_loc

        @pl.loop(0, _NQ)
        def _(q):
            base = q * _CH

            @pl.loop(0, _NZ)
            def _(z):
                pltpu.sync_copy(
                    zero_v, acc_sh.at[pl.ds(sid * _STRIPE + z * _ZB, _ZB)]
                )

            plsc.subcore_barrier()

            base_vec = jnp.full((_L,), base, jnp.int32)

            @pl.loop(0, _NSTEP)
            def _(st):
                off = b * _N + sid * _P + st * _K
                pltpu.sync_copy(m_hbm.at[pl.ds(off, _K)], mask_v)
                pltpu.sync_copy(x_hbm.at[pl.ds(off, _K)], vals_v)

                @plsc.parallel_loop(0, _K // _L, unroll=4)
                def _(i):
                    m16 = mask_v[pl.ds(i * _L, _L)]
                    mu = plsc.bitcast(m16, jnp.uint32)
                    rel = plsc.bitcast(m16 - base_vec, jnp.uint32)
                    trash = ch_vec + (mu & tmask_vec)
                    idx = jnp.minimum(rel, trash)
                    idx_v[pl.ds(i * _L, _L)] = plsc.bitcast(idx, jnp.int32)

                pltpu.sync_copy(vals_v, acc_sh.at[idx_v], add=True)

            plsc.subcore_barrier()

            out_off = b * _M + base + sid * _STRIPE
            pltpu.sync_copy(
                acc_sh.at[pl.ds(sid * _STRIPE, _STRIPE)],
                out_hbm.at[pl.ds(out_off, _STRIPE)],
            )


@jax.jit
def kernel(input, mask):
    x = input.reshape(-1)
    m = mask.reshape(-1)
    mesh = plsc.VectorSubcoreMesh(core_axis_name="c", subcore_axis_name="s")
    out = pl.kernel(
        _unpool_body,
        out_type=jax.ShapeDtypeStruct((_B * _M,), jnp.float32),
        mesh=mesh,
        scratch_types=[
            pltpu.VMEM((_K,), jnp.int32),
            pltpu.VMEM((_K,), jnp.float32),
            pltpu.VMEM((_K,), jnp.int32),
            pltpu.VMEM((_ZB,), jnp.float32),
            pltpu.VMEM_SHARED((_CH + _TRASH,), jnp.float32),
        ],
    )(x, m)
    return out.reshape(_B, _H * _STRIDE, _W * _STRIDE, _C)


# speedup vs baseline: 1.2048x; 1.2048x over previous
import jax
import jax.numpy as jnp
from jax import lax
from jax.experimental import pallas as pl
from jax.experimental.pallas import tpu as pltpu
from jax.experimental.pallas import tpu_sc as plsc

_STRIDE = 2
_B, _H, _W, _C = 8, 112, 112, 96
_N = _H * _W * _C
_M = _N * _STRIDE * _STRIDE
_NC, _NS, _L = 2, 16, 16
_NQ = 4
_CH = _M // _NQ
_TRASH = 4096
_P = _N // _NS
_NSTEP = 12
_K = _P // _NSTEP
_STRIPE = _CH // _NS
_NZ = 16
_ZB = _STRIPE // _NZ
_BPC = _B // _NC


def _unpool_body(x_hbm, m_hbm, out_hbm,
                 mask0, vals0, idx0, mask1, vals1, idx1, zero_v, acc_sh,
                 sem_in0, sem_in1):
    cid = lax.axis_index("c")
    sid = lax.axis_index("s")

    bufs = ((mask0, vals0, idx0, sem_in0),
            (mask1, vals1, idx1, sem_in1))

    def start_in(s, off):
        mk, vl, _, si = bufs[s]
        pltpu.async_copy(m_hbm.at[pl.ds(off, _K)], mk, si)
        pltpu.async_copy(x_hbm.at[pl.ds(off, _K)], vl, si)

    def wait_in(s):
        mk, vl, _, si = bufs[s]
        pltpu.make_async_copy(m_hbm.at[pl.ds(0, _K)], mk, si).wait()
        pltpu.make_async_copy(x_hbm.at[pl.ds(0, _K)], vl, si).wait()

    def scatter(s):
        _, vl, ix, _ = bufs[s]
        pltpu.sync_copy(vl, acc_sh.at[ix], add=True)

    z16 = jnp.zeros((_L,), jnp.float32)

    @pl.loop(0, _ZB // _L)
    def _(i):
        zero_v[pl.ds(i * _L, _L)] = z16

    ch_vec = jnp.full((_L,), _CH, jnp.uint32)
    tmask_vec = jnp.full((_L,), _TRASH - 1, jnp.uint32)

    def compute_idx(s, base_vec):
        mk, _, ix, _ = bufs[s]

        @plsc.parallel_loop(0, _K // _L, unroll=4)
        def _(i):
            m16 = mk[pl.ds(i * _L, _L)]
            mu = plsc.bitcast(m16, jnp.uint32)
            rel = plsc.bitcast(m16 - base_vec, jnp.uint32)
            trash = ch_vec + (mu & tmask_vec)
            idx = jnp.minimum(rel, trash)
            ix[pl.ds(i * _L, _L)] = plsc.bitcast(idx, jnp.int32)

    @pl.loop(0, _BPC)
    def _(b_loc):
        b = cid * _BPC + b_loc
        pair0 = b * _N + sid * _P

        @pl.loop(0, _NQ)
        def _(q):
            base = q * _CH

            start_in(0, pair0)

            @pl.loop(0, _NZ)
            def _(z):
                pltpu.sync_copy(
                    zero_v, acc_sh.at[pl.ds(sid * _STRIPE + z * _ZB, _ZB)]
                )

            plsc.subcore_barrier()

            base_vec = jnp.full((_L,), base, jnp.int32)

            @pl.loop(0, _NSTEP // 2)
            def _(g):
                for sb in (0, 1):
                    st = 2 * g + sb
                    wait_in(sb)
                    compute_idx(sb, base_vec)
                    nxt = st + 1

                    @pl.when(nxt < _NSTEP)
                    def _():
                        start_in(1 - sb, pair0 + nxt * _K)

                    scatter(sb)

            plsc.subcore_barrier()

            out_off = b * _M + base + sid * _STRIPE
            pltpu.sync_copy(
                acc_sh.at[pl.ds(sid * _STRIPE, _STRIPE)],
                out_hbm.at[pl.ds(out_off, _STRIPE)],
            )


@jax.jit
def kernel(input, mask):
    x = input.reshape(-1)
    m = mask.reshape(-1)
    mesh = plsc.VectorSubcoreMesh(core_axis_name="c", subcore_axis_name="s")
    out = pl.kernel(
        _unpool_body,
        out_type=jax.ShapeDtypeStruct((_B * _M,), jnp.float32),
        mesh=mesh,
        scratch_types=[
            pltpu.VMEM((_K,), jnp.int32),
            pltpu.VMEM((_K,), jnp.float32),
            pltpu.VMEM((_K,), jnp.int32),
            pltpu.VMEM((_K,), jnp.int32),
            pltpu.VMEM((_K,), jnp.float32),
            pltpu.VMEM((_K,), jnp.int32),
            pltpu.VMEM((_ZB,), jnp.float32),
            pltpu.VMEM_SHARED((_CH + _TRASH,), jnp.float32),
            pltpu.SemaphoreType.DMA,
            pltpu.SemaphoreType.DMA,
        ],
    )(x, m)
    return out.reshape(_B, _H * _STRIDE, _W * _STRIDE, _C)


# speedup vs baseline: 1.3191x; 1.0948x over previous
import jax
import jax.numpy as jnp
from jax import lax
from jax.experimental import pallas as pl
from jax.experimental.pallas import tpu as pltpu
from jax.experimental.pallas import tpu_sc as plsc

_STRIDE = 2
_B, _H, _W, _C = 8, 112, 112, 96
_N = _H * _W * _C
_M = _N * _STRIDE * _STRIDE
_NC, _NS, _L = 2, 16, 16
_NQ = 3
_CH = _M // _NQ
_TRASH = 4096
_P = _N // _NS
_NSTEP = 24
_K = _P // _NSTEP
_STRIPE = _CH // _NS
_NZ = 16
_ZB = _STRIPE // _NZ
_BPC = _B // _NC


def _unpool_body(x_hbm, m_hbm, out_hbm,
                 mask0, vals0, idx0, mask1, vals1, idx1, zero_v, acc_sh,
                 sem_in0, sem_in1):
    cid = lax.axis_index("c")
    sid = lax.axis_index("s")

    bufs = ((mask0, vals0, idx0, sem_in0),
            (mask1, vals1, idx1, sem_in1))

    def start_in(s, off):
        mk, vl, _, si = bufs[s]
        pltpu.async_copy(m_hbm.at[pl.ds(off, _K)], mk, si)
        pltpu.async_copy(x_hbm.at[pl.ds(off, _K)], vl, si)

    def wait_in(s):
        mk, vl, _, si = bufs[s]
        pltpu.make_async_copy(m_hbm.at[pl.ds(0, _K)], mk, si).wait()
        pltpu.make_async_copy(x_hbm.at[pl.ds(0, _K)], vl, si).wait()

    def scatter(s):
        _, vl, ix, _ = bufs[s]
        pltpu.sync_copy(vl, acc_sh.at[ix], add=True)

    z16 = jnp.zeros((_L,), jnp.float32)

    @pl.loop(0, _ZB // _L)
    def _(i):
        zero_v[pl.ds(i * _L, _L)] = z16

    ch_vec = jnp.full((_L,), _CH, jnp.uint32)
    tmask_vec = jnp.full((_L,), _TRASH - 1, jnp.uint32)

    def compute_idx(s, base_vec):
        mk, _, ix, _ = bufs[s]

        @plsc.parallel_loop(0, _K // _L, unroll=4)
        def _(i):
            m16 = mk[pl.ds(i * _L, _L)]
            mu = plsc.bitcast(m16, jnp.uint32)
            rel = plsc.bitcast(m16 - base_vec, jnp.uint32)
            trash = ch_vec + (mu & tmask_vec)
            idx = jnp.minimum(rel, trash)
            ix[pl.ds(i * _L, _L)] = plsc.bitcast(idx, jnp.int32)

    @pl.loop(0, _BPC)
    def _(b_loc):
        b = cid * _BPC + b_loc
        pair0 = b * _N + sid * _P

        @pl.loop(0, _NQ)
        def _(q):
            base = q * _CH

            start_in(0, pair0)

            @pl.loop(0, _NZ)
            def _(z):
                pltpu.sync_copy(
                    zero_v, acc_sh.at[pl.ds(sid * _STRIPE + z * _ZB, _ZB)]
                )

            plsc.subcore_barrier()

            base_vec = jnp.full((_L,), base, jnp.int32)

            @pl.loop(0, _NSTEP // 2)
            def _(g):
                for sb in (0, 1):
                    st = 2 * g + sb
                    wait_in(sb)
                    compute_idx(sb, base_vec)
                    nxt = st + 1

                    @pl.when(nxt < _NSTEP)
                    def _():
                        start_in(1 - sb, pair0 + nxt * _K)

                    scatter(sb)

            plsc.subcore_barrier()

            out_off = b * _M + base + sid * _STRIPE
            pltpu.sync_copy(
                acc_sh.at[pl.ds(sid * _STRIPE, _STRIPE)],
                out_hbm.at[pl.ds(out_off, _STRIPE)],
            )


@jax.jit
def kernel(input, mask):
    x = input.reshape(-1)
    m = mask.reshape(-1)
    mesh = plsc.VectorSubcoreMesh(core_axis_name="c", subcore_axis_name="s")
    out = pl.kernel(
        _unpool_body,
        out_type=jax.ShapeDtypeStruct((_B * _M,), jnp.float32),
        mesh=mesh,
        scratch_types=[
            pltpu.VMEM((_K,), jnp.int32),
            pltpu.VMEM((_K,), jnp.float32),
            pltpu.VMEM((_K,), jnp.int32),
            pltpu.VMEM((_K,), jnp.int32),
            pltpu.VMEM((_K,), jnp.float32),
            pltpu.VMEM((_K,), jnp.int32),
            pltpu.VMEM((_ZB,), jnp.float32),
            pltpu.VMEM_SHARED((_CH + _TRASH,), jnp.float32),
            pltpu.SemaphoreType.DMA,
            pltpu.SemaphoreType.DMA,
        ],
    )(x, m)
    return out.reshape(_B, _H * _STRIDE, _W * _STRIDE, _C)


# speedup vs baseline: 1.3194x; 1.0003x over previous
import jax
import jax.numpy as jnp
from jax import lax
from jax.experimental import pallas as pl
from jax.experimental.pallas import tpu as pltpu
from jax.experimental.pallas import tpu_sc as plsc

_STRIDE = 2
_B, _H, _W, _C = 8, 112, 112, 96
_N = _H * _W * _C
_M = _N * _STRIDE * _STRIDE
_NC, _NS, _L = 2, 16, 16
_NQ = 3
_CH = _M // _NQ
_TRASH = 4096
_P = _N // _NS
_NSTEP = 16
_K = _P // _NSTEP
_STRIPE = _CH // _NS
_NZ = 32
_ZB = _STRIPE // _NZ
_BPC = _B // _NC


def _unpool_body(x_hbm, m_hbm, out_hbm,
                 mask0, vals0, idx0, mask1, vals1, zero_v, acc_sh,
                 sem_in0, sem_in1):
    cid = lax.axis_index("c")
    sid = lax.axis_index("s")

    bufs = ((mask0, vals0, idx0, sem_in0),
            (mask1, vals1, idx0, sem_in1))

    def start_in(s, off):
        mk, vl, _, si = bufs[s]
        pltpu.async_copy(m_hbm.at[pl.ds(off, _K)], mk, si)
        pltpu.async_copy(x_hbm.at[pl.ds(off, _K)], vl, si)

    def wait_in(s):
        mk, vl, _, si = bufs[s]
        pltpu.make_async_copy(m_hbm.at[pl.ds(0, _K)], mk, si).wait()
        pltpu.make_async_copy(x_hbm.at[pl.ds(0, _K)], vl, si).wait()

    def scatter(s):
        _, vl, ix, _ = bufs[s]
        pltpu.sync_copy(vl, acc_sh.at[ix], add=True)

    z16 = jnp.zeros((_L,), jnp.float32)

    @pl.loop(0, _ZB // _L)
    def _(i):
        zero_v[pl.ds(i * _L, _L)] = z16

    ch_vec = jnp.full((_L,), _CH, jnp.uint32)
    tmask_vec = jnp.full((_L,), _TRASH - 1, jnp.uint32)

    def compute_idx(s, base_vec):
        mk, _, ix, _ = bufs[s]

        @plsc.parallel_loop(0, _K // _L, unroll=4)
        def _(i):
            m16 = mk[pl.ds(i * _L, _L)]
            mu = plsc.bitcast(m16, jnp.uint32)
            rel = plsc.bitcast(m16 - base_vec, jnp.uint32)
            trash = ch_vec + (mu & tmask_vec)
            idx = jnp.minimum(rel, trash)
            ix[pl.ds(i * _L, _L)] = plsc.bitcast(idx, jnp.int32)

    @pl.loop(0, _BPC)
    def _(b_loc):
        b = cid * _BPC + b_loc
        pair0 = b * _N + sid * _P

        @pl.loop(0, _NQ)
        def _(q):
            base = q * _CH

            start_in(0, pair0)

            @pl.loop(0, _NZ)
            def _(z):
                pltpu.sync_copy(
                    zero_v, acc_sh.at[pl.ds(sid * _STRIPE + z * _ZB, _ZB)]
                )

            plsc.subcore_barrier()

            base_vec = jnp.full((_L,), base, jnp.int32)

            @pl.loop(0, _NSTEP // 2)
            def _(g):
                for sb in (0, 1):
                    st = 2 * g + sb
                    wait_in(sb)
                    compute_idx(sb, base_vec)
                    nxt = st + 1

                    @pl.when(nxt < _NSTEP)
                    def _():
                        start_in(1 - sb, pair0 + nxt * _K)

                    scatter(sb)

            plsc.subcore_barrier()

            out_off = b * _M + base + sid * _STRIPE
            pltpu.sync_copy(
                acc_sh.at[pl.ds(sid * _STRIPE, _STRIPE)],
                out_hbm.at[pl.ds(out_off, _STRIPE)],
            )


@jax.jit
def kernel(input, mask):
    x = input.reshape(-1)
    m = mask.reshape(-1)
    mesh = plsc.VectorSubcoreMesh(core_axis_name="c", subcore_axis_name="s")
    out = pl.kernel(
        _unpool_body,
        out_type=jax.ShapeDtypeStruct((_B * _M,), jnp.float32),
        mesh=mesh,
        scratch_types=[
            pltpu.VMEM((_K,), jnp.int32),
            pltpu.VMEM((_K,), jnp.float32),
            pltpu.VMEM((_K,), jnp.int32),
            pltpu.VMEM((_K,), jnp.int32),
            pltpu.VMEM((_K,), jnp.float32),
            pltpu.VMEM((_ZB,), jnp.float32),
            pltpu.VMEM_SHARED((_CH + _TRASH,), jnp.float32),
            pltpu.SemaphoreType.DMA,
            pltpu.SemaphoreType.DMA,
        ],
    )(x, m)
    return out.reshape(_B, _H * _STRIDE, _W * _STRIDE, _C)


# speedup vs baseline: 1.3315x; 1.0092x over previous
import jax
import jax.numpy as jnp
from jax import lax
from jax.experimental import pallas as pl
from jax.experimental.pallas import tpu as pltpu
from jax.experimental.pallas import tpu_sc as plsc

_STRIDE = 2
_B, _H, _W, _C = 8, 112, 112, 96
_N = _H * _W * _C
_M = _N * _STRIDE * _STRIDE
_NC, _NS, _L = 2, 16, 16
_NQ = 3
_CH = _M // _NQ
_TRASH = 4096
_P = _N // _NS
_NSTEP = 24
_K = _P // _NSTEP
_STRIPE = _CH // _NS
_NZ = 32
_ZB = _STRIPE // _NZ
_BPC = _B // _NC


def _unpool_body(x_hbm, m_hbm, out_hbm,
                 mask0, vals0, idx0, mask1, vals1, idx1, zero_v, acc_sh,
                 sem_in0, sem_in1, sem_sc0, sem_sc1):
    cid = lax.axis_index("c")
    sid = lax.axis_index("s")

    bufs = ((mask0, vals0, idx0, sem_in0, sem_sc0),
            (mask1, vals1, idx1, sem_in1, sem_sc1))

    def start_in(s, off):
        mk, vl, _, si, _ = bufs[s]
        pltpu.async_copy(m_hbm.at[pl.ds(off, _K)], mk, si)
        pltpu.async_copy(x_hbm.at[pl.ds(off, _K)], vl, si)

    def wait_in(s):
        mk, vl, _, si, _ = bufs[s]
        pltpu.make_async_copy(m_hbm.at[pl.ds(0, _K)], mk, si).wait()
        pltpu.make_async_copy(x_hbm.at[pl.ds(0, _K)], vl, si).wait()

    def start_sc(s):
        _, vl, ix, _, ss = bufs[s]
        pltpu.async_copy(vl, acc_sh.at[ix], ss, add=True)

    def wait_sc(s):
        _, vl, ix, _, ss = bufs[s]
        pltpu.make_async_copy(vl, acc_sh.at[ix], ss).wait()

    z16 = jnp.zeros((_L,), jnp.float32)

    @pl.loop(0, _ZB // _L)
    def _(i):
        zero_v[pl.ds(i * _L, _L)] = z16

    ch_vec = jnp.full((_L,), _CH, jnp.uint32)
    tmask_vec = jnp.full((_L,), _TRASH - 1, jnp.uint32)

    def compute_idx(s, base_vec):
        mk, _, ix, _, _ = bufs[s]

        @plsc.parallel_loop(0, _K // _L, unroll=4)
        def _(i):
            m16 = mk[pl.ds(i * _L, _L)]
            mu = plsc.bitcast(m16, jnp.uint32)
            rel = plsc.bitcast(m16 - base_vec, jnp.uint32)
            trash = ch_vec + (mu & tmask_vec)
            idx = jnp.minimum(rel, trash)
            ix[pl.ds(i * _L, _L)] = plsc.bitcast(idx, jnp.int32)

    @pl.loop(0, _BPC)
    def _(b_loc):
        b = cid * _BPC + b_loc
        pair0 = b * _N + sid * _P

        @pl.loop(0, _NQ)
        def _(q):
            base = q * _CH

            start_in(0, pair0)

            @pl.loop(0, _NZ)
            def _(z):
                pltpu.sync_copy(
                    zero_v, acc_sh.at[pl.ds(sid * _STRIPE + z * _ZB, _ZB)]
                )

            plsc.subcore_barrier()

            base_vec = jnp.full((_L,), base, jnp.int32)

            @pl.loop(0, _NSTEP // 2)
            def _(g):
                for sb in (0, 1):
                    st = 2 * g + sb
                    wait_in(sb)
                    compute_idx(sb, base_vec)
                    start_sc(sb)
                    nxt = st + 1

                    @pl.when(nxt < _NSTEP)
                    def _():
                        @pl.when(st >= 1)
                        def _():
                            wait_sc(1 - sb)

                        start_in(1 - sb, pair0 + nxt * _K)

            wait_sc(0)
            wait_sc(1)
            plsc.subcore_barrier()

            out_off = b * _M + base + sid * _STRIPE
            pltpu.sync_copy(
                acc_sh.at[pl.ds(sid * _STRIPE, _STRIPE)],
                out_hbm.at[pl.ds(out_off, _STRIPE)],
            )


@jax.jit
def kernel(input, mask):
    x = input.reshape(-1)
    m = mask.reshape(-1)
    mesh = plsc.VectorSubcoreMesh(core_axis_name="c", subcore_axis_name="s")
    out = pl.kernel(
        _unpool_body,
        out_type=jax.ShapeDtypeStruct((_B * _M,), jnp.float32),
        mesh=mesh,
        scratch_types=[
            pltpu.VMEM((_K,), jnp.int32),
            pltpu.VMEM((_K,), jnp.float32),
            pltpu.VMEM((_K,), jnp.int32),
            pltpu.VMEM((_K,), jnp.int32),
            pltpu.VMEM((_K,), jnp.float32),
            pltpu.VMEM((_K,), jnp.int32),
            pltpu.VMEM((_ZB,), jnp.float32),
            pltpu.VMEM_SHARED((_CH + _TRASH,), jnp.float32),
            pltpu.SemaphoreType.DMA,
            pltpu.SemaphoreType.DMA,
            pltpu.SemaphoreType.DMA,
            pltpu.SemaphoreType.DMA,
        ],
    )(x, m)
    return out.reshape(_B, _H * _STRIDE, _W * _STRIDE, _C)


# speedup vs baseline: 1.3411x; 1.0072x over previous
import jax
import jax.numpy as jnp
from jax import lax
from jax.experimental import pallas as pl
from jax.experimental.pallas import tpu as pltpu
from jax.experimental.pallas import tpu_sc as plsc

_STRIDE = 2
_B, _H, _W, _C = 8, 112, 112, 96
_N = _H * _W * _C
_M = _N * _STRIDE * _STRIDE
_NC, _NS, _L = 2, 16, 16
_NQ = 3
_CH = _M // _NQ
_TRASH = 4096
_P = _N // _NS
_NSTEP = 24
_K = _P // _NSTEP
_STRIPE = _CH // _NS
_NZ = 32
_ZB = _STRIPE // _NZ
_BPC = _B // _NC


def _unpool_body(x_hbm, m_hbm, out_hbm,
                 mask0, vals0, idx0, mask1, vals1, idx1, zero_v, acc_sh,
                 sem_in0, sem_in1, sem_sc0, sem_sc1):
    cid = lax.axis_index("c")
    sid = lax.axis_index("s")

    bufs = ((mask0, vals0, idx0, sem_in0, sem_sc0),
            (mask1, vals1, idx1, sem_in1, sem_sc1))

    def start_in(s, off):
        mk, vl, _, si, _ = bufs[s]
        pltpu.async_copy(m_hbm.at[pl.ds(off, _K)], mk, si)
        pltpu.async_copy(x_hbm.at[pl.ds(off, _K)], vl, si)

    def wait_in(s):
        mk, vl, _, si, _ = bufs[s]
        pltpu.make_async_copy(m_hbm.at[pl.ds(0, _K)], mk, si).wait()
        pltpu.make_async_copy(x_hbm.at[pl.ds(0, _K)], vl, si).wait()

    def start_sc(s):
        _, vl, ix, _, ss = bufs[s]
        pltpu.async_copy(vl, acc_sh.at[ix], ss, add=True)

    def wait_sc(s):
        _, vl, ix, _, ss = bufs[s]
        pltpu.make_async_copy(vl, acc_sh.at[ix], ss).wait()

    z16 = jnp.zeros((_L,), jnp.float32)

    @pl.loop(0, _ZB // _L)
    def _(i):
        zero_v[pl.ds(i * _L, _L)] = z16

    ch_vec = jnp.full((_L,), _CH, jnp.uint32)
    tmask_vec = jnp.full((_L,), _TRASH - 1, jnp.uint32)

    def compute_idx(s, base_vec):
        mk, _, ix, _, _ = bufs[s]

        @plsc.parallel_loop(0, _K // _L, unroll=8)
        def _(i):
            m16 = mk[pl.ds(i * _L, _L)]
            mu = plsc.bitcast(m16, jnp.uint32)
            rel = plsc.bitcast(m16 - base_vec, jnp.uint32)
            trash = ch_vec + (mu & tmask_vec)
            idx = jnp.minimum(rel, trash)
            ix[pl.ds(i * _L, _L)] = plsc.bitcast(idx, jnp.int32)

    @pl.loop(0, _BPC)
    def _(b_loc):
        b = cid * _BPC + b_loc
        pair0 = b * _N + sid * _P

        @pl.loop(0, _NQ)
        def _(q):
            base = q * _CH

            start_in(0, pair0)

            @pl.loop(0, _NZ)
            def _(z):
                pltpu.sync_copy(
                    zero_v, acc_sh.at[pl.ds(sid * _STRIPE + z * _ZB, _ZB)]
                )

            plsc.subcore_barrier()

            base_vec = jnp.full((_L,), base, jnp.int32)

            @pl.loop(0, _NSTEP // 2)
            def _(g):
                for sb in (0, 1):
                    st = 2 * g + sb
                    wait_in(sb)
                    compute_idx(sb, base_vec)
                    start_sc(sb)
                    nxt = st + 1

                    @pl.when(nxt < _NSTEP)
                    def _():
                        @pl.when(st >= 1)
                        def _():
                            wait_sc(1 - sb)

                        start_in(1 - sb, pair0 + nxt * _K)

            wait_sc(0)
            wait_sc(1)
            plsc.subcore_barrier()

            out_off = b * _M + base + sid * _STRIPE
            pltpu.sync_copy(
                acc_sh.at[pl.ds(sid * _STRIPE, _STRIPE)],
                out_hbm.at[pl.ds(out_off, _STRIPE)],
            )


@jax.jit
def kernel(input, mask):
    x = input.reshape(-1)
    m = mask.reshape(-1)
    mesh = plsc.VectorSubcoreMesh(core_axis_name="c", subcore_axis_name="s")
    out = pl.kernel(
        _unpool_body,
        out_type=jax.ShapeDtypeStruct((_B * _M,), jnp.float32),
        mesh=mesh,
        scratch_types=[
            pltpu.VMEM((_K,), jnp.int32),
            pltpu.VMEM((_K,), jnp.float32),
            pltpu.VMEM((_K,), jnp.int32),
            pltpu.VMEM((_K,), jnp.int32),
            pltpu.VMEM((_K,), jnp.float32),
            pltpu.VMEM((_K,), jnp.int32),
            pltpu.VMEM((_ZB,), jnp.float32),
            pltpu.VMEM_SHARED((_CH + _TRASH,), jnp.float32),
            pltpu.SemaphoreType.DMA,
            pltpu.SemaphoreType.DMA,
            pltpu.SemaphoreType.DMA,
            pltpu.SemaphoreType.DMA,
        ],
    )(x, m)
    return out.reshape(_B, _H * _STRIDE, _W * _STRIDE, _C)
